# baseline (speedup 1.0000x reference)
import jax
import jax.numpy as jnp
from jax.experimental import pallas as pl
from jax.experimental.pallas import tpu as pltpu

_NSLOT = 3


def _fused_kernel(x_ref, w_ref, adj_hbm, out_ref, s_ref, abuf, sems):
    i = pl.program_id(0)
    j = pl.program_id(1)
    ni = pl.num_programs(0)
    nj = pl.num_programs(1)
    bm = abuf.shape[1]
    bj = x_ref.shape[0]
    k = x_ref.shape[1]
    f_out = w_ref.shape[1]

    def start_copy(slab):
        slot = jax.lax.rem(slab, _NSLOT)
        pltpu.make_async_copy(
            adj_hbm.at[pl.ds(slab * bm, bm), :],
            abuf.at[slot],
            sems.at[slot],
        ).start()

    @pl.when(jnp.logical_and(i == 0, j == 0))
    def _prime():
        start_copy(jnp.int32(0))
        start_copy(jnp.int32(1))

    @pl.when(jnp.logical_and(j == 0, i + 2 < ni))
    def _ahead():
        start_copy(i + 2)

    @pl.when(i == 0)
    def _fill():
        w = w_ref[...]
        for c in range(k):
            s_ref[pl.ds(j * bj, bj), c * f_out:(c + 1) * f_out] = jnp.dot(
                x_ref[:, c, :], w, preferred_element_type=jnp.float32)

    @pl.when(j == nj - 1)
    def _compute():
        slot = jax.lax.rem(i, _NSLOT)
        pltpu.make_async_copy(
            adj_hbm.at[pl.ds(i * bm, bm), :],
            abuf.at[slot],
            sems.at[slot],
        ).wait()
        acc = jnp.dot(abuf[slot], s_ref[...],
                      preferred_element_type=jnp.float32)
        out_ref[...] = jnp.maximum(acc, 0.0)


def kernel(input, adj, weight):
    n, k, f_in = input.shape
    f_out = weight.shape[1]
    bm = 200
    bj = 2000

    out2d = pl.pallas_call(
        _fused_kernel,
        grid=(n // bm, n // bj),
        in_specs=[
            pl.BlockSpec((bj, k, f_in),
                         lambda i, j: (jnp.where(i == 0, j, 0), 0, 0)),
            pl.BlockSpec((f_in, f_out), lambda i, j: (0, 0)),
            pl.BlockSpec(memory_space=pl.MemorySpace.ANY),
        ],
        out_specs=pl.BlockSpec((bm, k * f_out), lambda i, j: (i, 0)),
        out_shape=jax.ShapeDtypeStruct((n, k * f_out), jnp.float32),
        scratch_shapes=[
            pltpu.VMEM((n, k * f_out), jnp.float32),
            pltpu.VMEM((_NSLOT, bm, n), jnp.float32),
            pltpu.SemaphoreType.DMA((_NSLOT,)),
        ],
    )(input, weight, adj)
    return out2d.reshape(n, k, f_out)


# speedup vs baseline: 1.1397x; 1.1397x over previous
import jax
import jax.numpy as jnp
from jax.experimental import pallas as pl
from jax.experimental.pallas import tpu as pltpu


def _fused_kernel(x_ref, w_ref, adj_a, adj_b, out_ref, s_ref):
    i = pl.program_id(0)
    j = pl.program_id(1)
    nj = pl.num_programs(1)
    bj = x_ref.shape[0]
    k = x_ref.shape[1]
    f_out = w_ref.shape[1]
    h = adj_a.shape[0]

    @pl.when(i == 0)
    def _fill():
        w = w_ref[...]
        for c in range(k):
            s_ref[pl.ds(j * bj, bj), c * f_out:(c + 1) * f_out] = jnp.dot(
                x_ref[:, c, :], w,
                preferred_element_type=jnp.float32).astype(jnp.bfloat16)

    @pl.when(j == nj - 1)
    def _compute():
        s = s_ref[...]
        out_ref[:h, :] = jnp.maximum(
            jnp.dot(adj_a[...].astype(jnp.bfloat16), s,
                    preferred_element_type=jnp.float32), 0.0)
        out_ref[h:, :] = jnp.maximum(
            jnp.dot(adj_b[...].astype(jnp.bfloat16), s,
                    preferred_element_type=jnp.float32), 0.0)


def kernel(input, adj, weight):
    n, k, f_in = input.shape
    f_out = weight.shape[1]
    bm = 400
    h = bm // 2
    bj = 1000

    out2d = pl.pallas_call(
        _fused_kernel,
        grid=(n // bm, n // bj),
        in_specs=[
            pl.BlockSpec((bj, k, f_in),
                         lambda i, j: (jnp.where(i == 0, j, 0), 0, 0)),
            pl.BlockSpec((f_in, f_out), lambda i, j: (0, 0)),
            pl.BlockSpec((h, n), lambda i, j: (2 * i, 0)),
            pl.BlockSpec((h, n), lambda i, j: (2 * i + 1, 0)),
        ],
        out_specs=pl.BlockSpec((bm, k * f_out), lambda i, j: (i, 0)),
        out_shape=jax.ShapeDtypeStruct((n, k * f_out), jnp.float32),
        scratch_shapes=[pltpu.VMEM((n, k * f_out), jnp.bfloat16)],
    )(input, weight, adj, adj)
    return out2d.reshape(n, k, f_out)
